# baseline (speedup 1.0000x reference)
import functools

import jax
import jax.numpy as jnp
from jax import lax
from jax.experimental import pallas as pl
from jax.experimental.pallas import tpu as pltpu
from jax.experimental.pallas import tpu_sc as plsc

_VOCAB = 8192
_D = 8192
_B, _S = 8, 1024
_NTOK = _B * _S
_NC, _NS = 2, 16
_NW = _NC * _NS
_PER_W = _NTOK // _NW
_C = 4
_NCH = _PER_W // _C
_WPB = _S // _PER_W
_NBUF = 3


def _body(idx_hbm, table_hbm, out_hbm, idx_v, b0, b1, b2,
          gs0, gs1, gs2, os0, os1, os2):
    wid = lax.axis_index("s") * _NC + lax.axis_index("c")
    batch = wid // _WPB
    s0 = (wid % _WPB) * _PER_W
    pltpu.sync_copy(idx_hbm.at[wid], idx_v)

    bufs = (b0, b1, b2)
    gsems = (gs0, gs1, gs2)
    osems = (os0, os1, os2)

    def g_start(i):
        s = i % _NBUF
        pltpu.async_copy(table_hbm.at[idx_v.at[i]], bufs[s], gsems[s])

    def g_wait(i):
        s = i % _NBUF
        pltpu.make_async_copy(table_hbm.at[idx_v.at[i]], bufs[s], gsems[s]).wait()

    def o_start(i):
        s = i % _NBUF
        pltpu.async_copy(
            bufs[s], out_hbm.at[batch, pl.ds(s0 + i * _C, _C)], osems[s])

    def o_wait(i):
        s = i % _NBUF
        pltpu.make_async_copy(
            bufs[s], out_hbm.at[batch, pl.ds(s0 + i * _C, _C)], osems[s]).wait()

    g_start(0)
    for k in range(_NCH):
        g_wait(k)
        o_start(k)
        if k >= 2:
            o_wait(k - 2)
        if k + 1 < _NCH:
            g_start(k + 1)
    o_wait(_NCH - 2)
    o_wait(_NCH - 1)


_gather = functools.partial(
    pl.kernel,
    out_type=jax.ShapeDtypeStruct((_B, _S, _D), jnp.float32),
    mesh=plsc.VectorSubcoreMesh(core_axis_name="c", subcore_axis_name="s"),
    scratch_types=[
        pltpu.VMEM((_NCH, _C), jnp.int32),
        pltpu.VMEM((_C, _D), jnp.float32),
        pltpu.VMEM((_C, _D), jnp.float32),
        pltpu.VMEM((_C, _D), jnp.float32),
        pltpu.SemaphoreType.DMA,
        pltpu.SemaphoreType.DMA,
        pltpu.SemaphoreType.DMA,
        pltpu.SemaphoreType.DMA,
        pltpu.SemaphoreType.DMA,
        pltpu.SemaphoreType.DMA,
    ],
)(_body)


def kernel(x, table):
    idx3 = x.reshape(_NW, _NCH, _C).astype(jnp.int32)
    return _gather(idx3, table)


# speedup vs baseline: 1.0019x; 1.0019x over previous
import functools

import jax
import jax.numpy as jnp
from jax import lax
from jax.experimental import pallas as pl
from jax.experimental.pallas import tpu as pltpu
from jax.experimental.pallas import tpu_sc as plsc

_VOCAB = 8192
_D = 8192
_B, _S = 8, 1024
_NTOK = _B * _S
_NC, _NS = 2, 16
_NW = _NC * _NS
_PER_W = _NTOK // _NW
_C = 2
_NCH = _PER_W // _C
_WPB = _S // _PER_W
_NBUF = 6
_GA = 3
_OW = 3


def _body(idx_hbm, table_hbm, out_hbm, idx_v, *rest):
    bufs = rest[:_NBUF]
    gsems = rest[_NBUF:2 * _NBUF]
    osems = rest[2 * _NBUF:]
    wid = lax.axis_index("s") * _NC + lax.axis_index("c")
    batch = wid // _WPB
    s0 = (wid % _WPB) * _PER_W
    pltpu.sync_copy(idx_hbm.at[wid], idx_v)

    def g_start(i):
        s = i % _NBUF
        pltpu.async_copy(table_hbm.at[idx_v.at[i]], bufs[s], gsems[s])

    def g_wait(i):
        s = i % _NBUF
        pltpu.make_async_copy(table_hbm.at[idx_v.at[i]], bufs[s], gsems[s]).wait()

    def o_start(i):
        s = i % _NBUF
        pltpu.async_copy(
            bufs[s], out_hbm.at[batch, pl.ds(s0 + i * _C, _C)], osems[s])

    def o_wait(i):
        s = i % _NBUF
        pltpu.make_async_copy(
            bufs[s], out_hbm.at[batch, pl.ds(s0 + i * _C, _C)], osems[s]).wait()

    for i in range(_GA):
        g_start(i)
    for k in range(_NCH):
        g_wait(k)
        o_start(k)
        if k >= _OW:
            o_wait(k - _OW)
        if k + _GA < _NCH:
            g_start(k + _GA)
    for i in range(_NCH - _OW, _NCH):
        o_wait(i)


_gather = functools.partial(
    pl.kernel,
    out_type=jax.ShapeDtypeStruct((_B, _S, _D), jnp.float32),
    mesh=plsc.VectorSubcoreMesh(core_axis_name="c", subcore_axis_name="s"),
    scratch_types=(
        [pltpu.VMEM((_NCH, _C), jnp.int32)]
        + [pltpu.VMEM((_C, _D), jnp.float32)] * _NBUF
        + [pltpu.SemaphoreType.DMA] * (2 * _NBUF)
    ),
)(_body)


def kernel(x, table):
    idx3 = x.reshape(_NW, _NCH, _C).astype(jnp.int32)
    return _gather(idx3, table)


# speedup vs baseline: 1.0090x; 1.0071x over previous
import functools

import jax
import jax.numpy as jnp
from jax import lax
from jax.experimental import pallas as pl
from jax.experimental.pallas import tpu as pltpu
from jax.experimental.pallas import tpu_sc as plsc

_VOCAB = 8192
_D = 8192
_B, _S = 8, 1024
_NTOK = _B * _S
_NC, _NS = 2, 16
_NW = _NC * _NS
_PER_W = _NTOK // _NW
_C = 4
_NCH = _PER_W // _C
_WPB = _S // _PER_W
_NBUF = 3


def _body(idx_hbm, table_hbm, out_hbm, idx_v, b0, b1, b2,
          gs0, gs1, gs2, os0, os1, os2):
    wid = lax.axis_index("s") * _NC + lax.axis_index("c")
    batch = wid // _WPB
    s0 = (wid % _WPB) * _PER_W
    pltpu.sync_copy(idx_hbm.at[wid], idx_v)

    bufs = (b0, b1, b2)
    gsems = (gs0, gs1, gs2)
    osems = (os0, os1, os2)

    def g_start(i):
        s = i % _NBUF
        pltpu.async_copy(table_hbm.at[idx_v.at[i]], bufs[s], gsems[s])

    def g_wait(i):
        s = i % _NBUF
        pltpu.make_async_copy(table_hbm.at[idx_v.at[i]], bufs[s], gsems[s]).wait()

    def o_start(i):
        s = i % _NBUF
        pltpu.async_copy(
            bufs[s], out_hbm.at[batch, pl.ds(s0 + i * _C, _C)], osems[s])

    def o_wait(i):
        s = i % _NBUF
        pltpu.make_async_copy(
            bufs[s], out_hbm.at[batch, pl.ds(s0 + i * _C, _C)], osems[s]).wait()

    g_start(0)
    g_start(1)
    for k in range(_NCH):
        g_wait(k)
        o_start(k)
        o_wait(k)
        if k + 2 < _NCH:
            g_start(k + 2)


_gather = functools.partial(
    pl.kernel,
    out_type=jax.ShapeDtypeStruct((_B, _S, _D), jnp.float32),
    mesh=plsc.VectorSubcoreMesh(core_axis_name="c", subcore_axis_name="s"),
    scratch_types=[
        pltpu.VMEM((_NCH, _C), jnp.int32),
        pltpu.VMEM((_C, _D), jnp.float32),
        pltpu.VMEM((_C, _D), jnp.float32),
        pltpu.VMEM((_C, _D), jnp.float32),
        pltpu.SemaphoreType.DMA,
        pltpu.SemaphoreType.DMA,
        pltpu.SemaphoreType.DMA,
        pltpu.SemaphoreType.DMA,
        pltpu.SemaphoreType.DMA,
        pltpu.SemaphoreType.DMA,
    ],
)(_body)


def kernel(x, table):
    idx3 = x.reshape(_NW, _NCH, _C).astype(jnp.int32)
    return _gather(idx3, table)
